# Initial kernel scaffold; baseline (speedup 1.0000x reference)
import jax
import jax.numpy as jnp
from jax.experimental import pallas as pl


def kernel(x, edge_index, features, batch, emb, bn_gamma, bn_beta, conv_w, conv_b, hidden_w, hidden_b, fc_w, fc_b):
    raise NotImplementedError("write your pallas kernel here")


# speedup vs baseline: 6.0405x; 6.0405x over previous
import functools

import jax
import jax.numpy as jnp
from jax import lax
from jax.experimental import pallas as pl
from jax.experimental.pallas import tpu as pltpu
from jax.experimental.pallas import tpu_sc as plsc

N = 10000
E = 320000
C = 128
HID = 256
NT = 10
NTYPES = 32
NF = 16
NG = 64
EPS = 1e-5

NCORES = 2
NSUB = 16
NWORK = NCORES * NSUB
CHUNK = 128
EPW = -(-(E // NWORK) // CHUNK) * CHUNK
EPAD = EPW * NWORK
NCHUNK = EPW // CHUNK
ACC_N = 10240
RPT = ACC_N // NSUB
DUMMY = N


def _spmm_body(zs_hbm, srcp, dstp, zeros_hbm, out_hbm,
               src_v, dst_v, rows_v, acc_sh, sem):
    c = lax.axis_index("c")
    s = lax.axis_index("s")
    w = c * NSUB + s
    pltpu.sync_copy(zeros_hbm, acc_sh.at[pl.ds(s * RPT, RPT)])
    plsc.subcore_barrier()
    base = w * EPW

    def body(i, carry):
        off = base + i * CHUNK
        pltpu.sync_copy(srcp.at[pl.ds(off, CHUNK)], src_v)
        pltpu.sync_copy(dstp.at[pl.ds(off, CHUNK)], dst_v)
        pltpu.async_copy(zs_hbm.at[src_v], rows_v, sem).wait()
        pltpu.sync_copy(rows_v, acc_sh.at[dst_v], add=True)
        return carry

    lax.fori_loop(0, NCHUNK, body, 0)
    plsc.subcore_barrier()
    pltpu.sync_copy(acc_sh.at[pl.ds(s * RPT, RPT)],
                    out_hbm.at[c, pl.ds(s * RPT, RPT)])


@functools.lru_cache(maxsize=None)
def _spmm_kernel():
    mesh = plsc.VectorSubcoreMesh(core_axis_name="c", subcore_axis_name="s")
    return pl.kernel(
        _spmm_body,
        out_type=jax.ShapeDtypeStruct((NCORES, ACC_N, C), jnp.float32),
        mesh=mesh,
        scratch_types=[
            pltpu.VMEM((CHUNK,), jnp.int32),
            pltpu.VMEM((CHUNK,), jnp.int32),
            pltpu.VMEM((CHUNK, C), jnp.float32),
            pltpu.VMEM_SHARED((ACC_N, C), jnp.float32),
            pltpu.SemaphoreType.DMA,
        ],
    )


def _spmm_call(zs, srcp, dstp, zeros_c):
    return _spmm_kernel()(zs, srcp, dstp, zeros_c)


def _bn_matmul(u, dinv, gamma, beta, w):
    mean = jnp.mean(u, axis=0, keepdims=True)
    var = jnp.mean(u * u, axis=0, keepdims=True) - mean * mean
    hbn = (u - mean) * lax.rsqrt(var + EPS) * gamma + beta
    return dinv * jnp.dot(hbn, w, preferred_element_type=jnp.float32)


def _tc0_body(x_ref, emb_ref, cnt_ref, g_ref, be_ref, w_ref,
              zs_ref, h0_ref, dinv_ref):
    xv = x_ref[...]
    oh = (xv == lax.broadcasted_iota(jnp.int32, (N, NTYPES), 1))
    h0 = jnp.dot(oh.astype(jnp.float32), emb_ref[...],
                 preferred_element_type=jnp.float32)
    deg = cnt_ref[0, :N, 0:1] + cnt_ref[1, :N, 0:1] + 1.0
    dinv = lax.rsqrt(deg)
    zs_ref[...] = _bn_matmul(h0, dinv, g_ref[...], be_ref[...], w_ref[...])
    h0_ref[...] = h0
    dinv_ref[...] = dinv


def _tc0_call(x2, emb, cnt, g, be, w):
    return pl.pallas_call(
        _tc0_body,
        out_shape=[
            jax.ShapeDtypeStruct((N, C), jnp.float32),
            jax.ShapeDtypeStruct((N, C), jnp.float32),
            jax.ShapeDtypeStruct((N, 1), jnp.float32),
        ],
    )(x2, emb, cnt, g, be, w)


def _tc_mid_body(residual, emit_before, s2_ref, zs_ref, dinv_ref, b_ref,
                 g_ref, be_ref, w_ref, *rest):
    if residual:
        before_ref = rest[0]
        rest = rest[1:]
    if emit_before:
        zso_ref, bo_ref = rest
    else:
        (zso_ref,) = rest
    dinv = dinv_ref[...]
    u = (s2_ref[0, :N, :] + s2_ref[1, :N, :] + zs_ref[...]) * dinv + b_ref[...]
    u = jnp.maximum(u, 0.0)
    if residual:
        u = u + before_ref[...]
    if emit_before:
        bo_ref[...] = u
    zso_ref[...] = _bn_matmul(u, dinv, g_ref[...], be_ref[...], w_ref[...])


def _tc_mid_call(s2, zs, dinv, b, g, be, w, before, emit_before):
    residual = before is not None
    outs = [jax.ShapeDtypeStruct((N, C), jnp.float32)]
    if emit_before:
        outs.append(jax.ShapeDtypeStruct((N, C), jnp.float32))
    args = [s2, zs, dinv, b, g, be, w]
    if residual:
        args.append(before)
    return pl.pallas_call(
        functools.partial(_tc_mid_body, residual, emit_before),
        out_shape=outs,
    )(*args)


def _tc_final_body(s2_ref, zs_ref, dinv_ref, b_ref, before_ref, batch_ref,
                   feat_ref, hw_ref, hb_ref, fcw_ref, fcb_ref, out_ref):
    dinv = dinv_ref[...]
    u = (s2_ref[0, :N, :] + s2_ref[1, :N, :] + zs_ref[...]) * dinv + b_ref[...]
    u = jnp.maximum(u, 0.0) + before_ref[...]
    bv = batch_ref[...]
    oh = (bv == lax.broadcasted_iota(jnp.int32, (N, NG), 1)).astype(jnp.float32)
    pooled = lax.dot_general(oh, u, (((0,), (0,)), ((), ())),
                             preferred_element_type=jnp.float32)
    counts = lax.dot_general(oh, jnp.ones((N, 1), jnp.float32),
                             (((0,), (0,)), ((), ())),
                             preferred_element_type=jnp.float32)
    mean = pooled / jnp.maximum(counts, 1.0)
    g = jnp.concatenate([mean, feat_ref[...]], axis=1)
    hid = jnp.maximum(
        jnp.dot(g, hw_ref[...], preferred_element_type=jnp.float32)
        + hb_ref[...], 0.0)
    out_ref[...] = (jnp.dot(hid, fcw_ref[...],
                            preferred_element_type=jnp.float32) + fcb_ref[...])


def _tc_final_call(s2, zs, dinv, b, before, batch2, feats, hw, hb, fcw, fcb):
    return pl.pallas_call(
        _tc_final_body,
        out_shape=jax.ShapeDtypeStruct((NG, C), jnp.float32),
    )(s2, zs, dinv, b, before, batch2, feats, hw, hb, fcw, fcb)


def kernel(x, edge_index, features, batch, emb, bn_gamma, bn_beta,
           conv_w, conv_b, hidden_w, hidden_b, fc_w, fc_b):
    src = edge_index[0].astype(jnp.int32)
    dst = edge_index[1].astype(jnp.int32)
    pad = EPAD - E
    srcp = jnp.concatenate([src, jnp.zeros((pad,), jnp.int32)])
    dstp = jnp.concatenate([dst, jnp.full((pad,), DUMMY, jnp.int32)])
    zeros_c = jnp.zeros((RPT, C), jnp.float32)
    ones_nc = jnp.ones((N, C), jnp.float32)
    x2 = x.reshape(N, 1).astype(jnp.int32)
    batch2 = batch.reshape(N, 1).astype(jnp.int32)
    gam = bn_gamma.reshape(8, 1, C)
    bet = bn_beta.reshape(8, 1, C)
    ws = conv_w.reshape(8, C, C)
    bs = conv_b.reshape(8, 1, C)
    hb = hidden_b.reshape(1, HID)
    fcw = jnp.pad(fc_w, ((0, 0), (0, C - NT)))
    fcb = jnp.pad(fc_b, (0, C - NT)).reshape(1, C)

    cnt = _spmm_call(ones_nc, srcp, dstp, zeros_c)
    zs, before, dinv = _tc0_call(x2, emb, cnt, gam[0], bet[0], ws[0])
    for k in range(8):
        s2 = _spmm_call(zs, srcp, dstp, zeros_c)
        if k < 7:
            res = (k % 2 == 1)
            emit = ((k + 1) % 2 == 0)
            outs = _tc_mid_call(s2, zs, dinv, bs[k], gam[k + 1], bet[k + 1],
                                ws[k + 1], before if res else None, emit)
            if emit:
                zs, before = outs
            else:
                (zs,) = outs
        else:
            out = _tc_final_call(s2, zs, dinv, bs[k], before, batch2,
                                 features, hidden_w, hb, fcw, fcb)
    return out[:, :NT]
